# baseline (speedup 1.0000x reference)
import functools

import jax
import jax.numpy as jnp
from jax import lax
from jax.experimental import pallas as pl
from jax.experimental.pallas import tpu as pltpu
from jax.experimental.pallas import tpu_sc as plsc

VOCAB = 1000000
EMBED = 64
BATCH = 16384
NFIELDS = 3

ROWS = BATCH * NFIELDS
NC, NS = 2, 16
NW = NC * NS
B_PER_W = ROWS // NW
CHUNK = 128
NCHUNK = B_PER_W // CHUNK

_sc_mesh = plsc.VectorSubcoreMesh(core_axis_name="c", subcore_axis_name="s")


@functools.partial(
    pl.kernel,
    mesh=_sc_mesh,
    out_type=jax.ShapeDtypeStruct((ROWS, EMBED), jnp.float32),
    scratch_types=[
        pltpu.VMEM((NCHUNK, CHUNK), jnp.int32),
        pltpu.VMEM((B_PER_W, EMBED), jnp.float32),
        pltpu.SemaphoreType.DMA,
    ],
    compiler_params=pltpu.CompilerParams(use_tc_tiling_on_sc=False),
)
def _gather_sc(idx_hbm, table_hbm, out_hbm, idx_v, rows_v, sem):
    wid = lax.axis_index("s") * NC + lax.axis_index("c")
    pltpu.sync_copy(idx_hbm.at[wid], idx_v)
    copies = [
        pltpu.async_copy(
            table_hbm.at[idx_v.at[j]],
            rows_v.at[pl.ds(j * CHUNK, CHUNK)],
            sem,
        )
        for j in range(NCHUNK)
    ]
    for c in copies:
        c.wait()
    pltpu.sync_copy(rows_v, out_hbm.at[pl.ds(wid * B_PER_W, B_PER_W)])


_BLK = 2048


def _mlp_body(x_ref, w1_ref, b1_ref, w2_ref, b2_ref, o_ref):
    x = x_ref[...]
    h = lax.dot_general(
        x, w1_ref[...], (((1,), (1,)), ((), ())),
        preferred_element_type=jnp.float32,
    ) + b1_ref[...]
    o_ref[...] = lax.dot_general(
        h, w2_ref[...], (((1,), (1,)), ((), ())),
        preferred_element_type=jnp.float32,
    ) + b2_ref[...]


_mlp_tc = pl.pallas_call(
    _mlp_body,
    grid=(BATCH // _BLK,),
    in_specs=[
        pl.BlockSpec((_BLK, NFIELDS * EMBED), lambda i: (i, 0)),
        pl.BlockSpec((EMBED, NFIELDS * EMBED), lambda i: (0, 0)),
        pl.BlockSpec((1, EMBED), lambda i: (0, 0)),
        pl.BlockSpec((12, EMBED), lambda i: (0, 0)),
        pl.BlockSpec((1, 12), lambda i: (0, 0)),
    ],
    out_specs=pl.BlockSpec((_BLK, 12), lambda i: (i, 0)),
    out_shape=jax.ShapeDtypeStruct((BATCH, 12), jnp.float32),
)


def kernel(demo, table, W1, b1, W2, b2):
    idx = demo.reshape(NW, NCHUNK, CHUNK)
    emb = _gather_sc(idx, table)
    emb = emb.reshape(BATCH, NFIELDS * EMBED)
    return _mlp_tc(emb, W1, b1.reshape(1, EMBED), W2, b2.reshape(1, 12))
